# baseline (speedup 1.0000x reference)
import functools

import jax
import jax.numpy as jnp
from jax import lax
from jax.experimental import pallas as pl
from jax.experimental.pallas import tpu as pltpu
from jax.experimental.pallas import tpu_sc as plsc

OUT_H = 7
OUT_W = 7
SCALE = 0.25
SR = 2

K_PER_BIN = OUT_H * OUT_W * SR * SR * 4 // (OUT_H * OUT_W)
J_PER_ROI = OUT_H * OUT_W * K_PER_BIN

N_PAD = 1024
NUM_WORKERS = 32
TOTAL_BINS = N_PAD * OUT_H * OUT_W
BINS_PER_WORKER = TOTAL_BINS // NUM_WORKERS
CHUNK_BINS = 8
CHUNKS_PER_WORKER = BINS_PER_WORKER // CHUNK_BINS


def _idxw_body(rois_ref, idx_ref, w_ref, *, H, W, HW):
    blk = rois_ref.shape[0]
    j = lax.broadcasted_iota(jnp.int32, (blk, J_PER_ROI), 1)
    ax = j % 2
    ay = (j // 2) % 2
    sx = (j // 4) % 2
    sy = (j // 8) % 2
    q = (j // 16) % OUT_W
    p = j // (16 * OUT_W)

    bidx = rois_ref[:, 0:1].astype(jnp.int32)
    x1 = rois_ref[:, 1:2] * SCALE
    y1 = rois_ref[:, 2:3] * SCALE
    x2 = rois_ref[:, 3:4] * SCALE
    y2 = rois_ref[:, 4:5] * SCALE
    roi_w = jnp.maximum(x2 - x1, 1.0)
    roi_h = jnp.maximum(y2 - y1, 1.0)
    jy = (p * SR + sy).astype(jnp.float32)
    jx = (q * SR + sx).astype(jnp.float32)
    y = y1 + (jy + 0.5) * (roi_h / (OUT_H * SR))
    x = x1 + (jx + 0.5) * (roi_w / (OUT_W * SR))

    vy = (y >= -1.0) & (y <= float(H))
    vx = (x >= -1.0) & (x <= float(W))
    yc = jnp.clip(y, 0.0, float(H - 1))
    xc = jnp.clip(x, 0.0, float(W - 1))
    yl = jnp.floor(yc)
    xl = jnp.floor(xc)
    ly = yc - yl
    lx = xc - xl
    yl_i = yl.astype(jnp.int32)
    xl_i = xl.astype(jnp.int32)
    yh_i = jnp.minimum(yl_i + 1, H - 1)
    xh_i = jnp.minimum(xl_i + 1, W - 1)

    ypt = jnp.where(ay == 1, yh_i, yl_i)
    xpt = jnp.where(ax == 1, xh_i, xl_i)
    wy = jnp.where(ay == 1, ly, 1.0 - ly)
    wx = jnp.where(ax == 1, lx, 1.0 - lx)
    valid = (vy & vx).astype(jnp.float32)

    idx_ref[...] = bidx * HW + ypt * W + xpt
    w_ref[...] = wy * wx * valid * (1.0 / (SR * SR))


def _compute_idx_w(rois_pad, H, W):
    blk = 128
    grid = N_PAD // blk
    return pl.pallas_call(
        functools.partial(_idxw_body, H=H, W=W, HW=H * W),
        grid=(grid,),
        in_specs=[pl.BlockSpec((blk, 5), lambda i: (i, 0))],
        out_specs=[
            pl.BlockSpec((blk, J_PER_ROI), lambda i: (i, 0)),
            pl.BlockSpec((blk, J_PER_ROI), lambda i: (i, 0)),
        ],
        out_shape=[
            jax.ShapeDtypeStruct((N_PAD, J_PER_ROI), jnp.int32),
            jax.ShapeDtypeStruct((N_PAD, J_PER_ROI), jnp.float32),
        ],
    )(rois_pad)


def _sc_body(idx_hbm, w_hbm, table_hbm, out_hbm,
             idx_v, w_v, rows0, rows1, out0, out1,
             sem_g0, sem_g1, sem_o0, sem_o1):
    nc = 2
    wid = lax.axis_index("s") * nc + lax.axis_index("c")
    base_bin = wid * BINS_PER_WORKER

    kpc = CHUNK_BINS * K_PER_BIN
    pltpu.sync_copy(idx_hbm.at[pl.ds(base_bin * K_PER_BIN, BINS_PER_WORKER * K_PER_BIN)], idx_v)
    pltpu.sync_copy(w_hbm.at[pl.ds(base_bin * K_PER_BIN, BINS_PER_WORKER * K_PER_BIN)], w_v)

    bufs = ((rows0, sem_g0, out0, sem_o0), (rows1, sem_g1, out1, sem_o1))

    pltpu.async_copy(table_hbm.at[idx_v.at[pl.ds(0, kpc)]], rows0, sem_g0)
    pltpu.async_copy(table_hbm.at[idx_v.at[pl.ds(kpc, kpc)]], rows1, sem_g1)

    def pair_body(g, _):
        for par in range(2):
            rows_v, sem_g, out_v, sem_o = bufs[par]
            ch = 2 * g + par
            bin0 = base_bin + ch * CHUNK_BINS
            pltpu.make_async_copy(table_hbm.at[idx_v.at[pl.ds(ch * kpc, kpc)]], rows_v, sem_g).wait()
            @pl.when(g > 0)
            def _():
                pltpu.make_async_copy(
                    out_v, out_hbm.at[pl.ds(bin0, CHUNK_BINS)], sem_o).wait()

            def bin_body(i, _):
                wv = w_v[pl.ds(ch * kpc + i * K_PER_BIN, K_PER_BIN)]
                acc = [jnp.zeros((16,), jnp.float32) for _ in range(16)]
                for k in range(K_PER_BIN):
                    wk = jnp.broadcast_to(wv[k], (16,))
                    r = i * K_PER_BIN + k
                    for g in range(8):
                        u = rows_v[r, pl.ds(g * 16, 16)]
                        a = lax.bitcast_convert_type(u << 16, jnp.float32)
                        b = lax.bitcast_convert_type(u, jnp.float32)
                        acc[2 * g] = acc[2 * g] + wk * a
                        acc[2 * g + 1] = acc[2 * g + 1] + wk * b
                for c in range(16):
                    out_v[i, pl.ds(c * 16, 16)] = acc[c]
                return 0

            lax.fori_loop(0, CHUNK_BINS, bin_body, 0)
            pltpu.async_copy(out_v, out_hbm.at[pl.ds(bin0, CHUNK_BINS)], sem_o)
            @pl.when(ch + 2 < CHUNKS_PER_WORKER)
            def _():
                pltpu.async_copy(table_hbm.at[idx_v.at[pl.ds((ch + 2) * kpc, kpc)]], rows_v, sem_g)
        return 0

    lax.fori_loop(0, CHUNKS_PER_WORKER // 2, pair_body, 0)
    last0 = base_bin + (CHUNKS_PER_WORKER - 2) * CHUNK_BINS
    last1 = base_bin + (CHUNKS_PER_WORKER - 1) * CHUNK_BINS
    pltpu.make_async_copy(out0, out_hbm.at[pl.ds(last0, CHUNK_BINS)], sem_o0).wait()
    pltpu.make_async_copy(out1, out_hbm.at[pl.ds(last1, CHUNK_BINS)], sem_o1).wait()


def _sc_gather(idx_flat, w_flat, table):
    C = table.shape[1] * 2
    mesh = plsc.VectorSubcoreMesh(core_axis_name="c", subcore_axis_name="s")
    run = pl.kernel(
        _sc_body,
        out_type=jax.ShapeDtypeStruct((TOTAL_BINS, C), jnp.float32),
        mesh=mesh,
        scratch_types=[
            pltpu.VMEM((BINS_PER_WORKER * K_PER_BIN,), jnp.int32),
            pltpu.VMEM((BINS_PER_WORKER * K_PER_BIN,), jnp.float32),
            pltpu.VMEM((CHUNK_BINS * K_PER_BIN, C // 2), jnp.int32),
            pltpu.VMEM((CHUNK_BINS * K_PER_BIN, C // 2), jnp.int32),
            pltpu.VMEM((CHUNK_BINS, C), jnp.float32),
            pltpu.VMEM((CHUNK_BINS, C), jnp.float32),
            pltpu.SemaphoreType.DMA,
            pltpu.SemaphoreType.DMA,
            pltpu.SemaphoreType.DMA,
            pltpu.SemaphoreType.DMA,
        ],
    )
    return run(idx_flat, w_flat, table)


def kernel(input, rois):
    B, C, H, W = input.shape
    N = rois.shape[0]
    nbins = OUT_H * OUT_W
    table = jnp.transpose(input, (0, 2, 3, 1)).reshape(B * H * W, C)
    table = lax.bitcast_convert_type(
        table.astype(jnp.bfloat16).reshape(B * H * W, C // 2, 2), jnp.int32)
    rois_pad = jnp.zeros((N_PAD, 5), jnp.float32).at[:N].set(rois.astype(jnp.float32))
    idx, w = _compute_idx_w(rois_pad, H, W)
    out_rows = _sc_gather(idx.reshape(-1), w.reshape(-1), table)
    out = out_rows.reshape(N_PAD, nbins, C // 32, 2, 16)
    out = jnp.transpose(out, (0, 2, 4, 3, 1)).reshape(N_PAD, C, nbins)
    return out[:N].reshape(N, C, OUT_H, OUT_W)


# speedup vs baseline: 1.1265x; 1.1265x over previous
import functools

import jax
import jax.numpy as jnp
from jax import lax
from jax.experimental import pallas as pl
from jax.experimental.pallas import tpu as pltpu
from jax.experimental.pallas import tpu_sc as plsc

OUT_H = 7
OUT_W = 7
SCALE = 0.25
SR = 2

K_PER_BIN = OUT_H * OUT_W * SR * SR * 4 // (OUT_H * OUT_W)
J_PER_ROI = OUT_H * OUT_W * K_PER_BIN

N_PAD = 1024
NUM_WORKERS = 32
TOTAL_BINS = N_PAD * OUT_H * OUT_W
BINS_PER_WORKER = TOTAL_BINS // NUM_WORKERS
CHUNK_BINS = 8
CHUNKS_PER_WORKER = BINS_PER_WORKER // CHUNK_BINS


def _idxw_body(rois_ref, idx_ref, w_ref, *, H, W, HW):
    blk = rois_ref.shape[0]
    j = lax.broadcasted_iota(jnp.int32, (blk, J_PER_ROI), 1)
    ax = j % 2
    ay = (j // 2) % 2
    sx = (j // 4) % 2
    sy = (j // 8) % 2
    q = (j // 16) % OUT_W
    p = j // (16 * OUT_W)

    bidx = rois_ref[:, 0:1].astype(jnp.int32)
    x1 = rois_ref[:, 1:2] * SCALE
    y1 = rois_ref[:, 2:3] * SCALE
    x2 = rois_ref[:, 3:4] * SCALE
    y2 = rois_ref[:, 4:5] * SCALE
    roi_w = jnp.maximum(x2 - x1, 1.0)
    roi_h = jnp.maximum(y2 - y1, 1.0)
    jy = (p * SR + sy).astype(jnp.float32)
    jx = (q * SR + sx).astype(jnp.float32)
    y = y1 + (jy + 0.5) * (roi_h / (OUT_H * SR))
    x = x1 + (jx + 0.5) * (roi_w / (OUT_W * SR))

    vy = (y >= -1.0) & (y <= float(H))
    vx = (x >= -1.0) & (x <= float(W))
    yc = jnp.clip(y, 0.0, float(H - 1))
    xc = jnp.clip(x, 0.0, float(W - 1))
    yl = jnp.floor(yc)
    xl = jnp.floor(xc)
    ly = yc - yl
    lx = xc - xl
    yl_i = yl.astype(jnp.int32)
    xl_i = xl.astype(jnp.int32)
    yh_i = jnp.minimum(yl_i + 1, H - 1)
    xh_i = jnp.minimum(xl_i + 1, W - 1)

    ypt = jnp.where(ay == 1, yh_i, yl_i)
    xpt = jnp.where(ax == 1, xh_i, xl_i)
    wy = jnp.where(ay == 1, ly, 1.0 - ly)
    wx = jnp.where(ax == 1, lx, 1.0 - lx)
    valid = (vy & vx).astype(jnp.float32)

    idx_ref[...] = bidx * HW + ypt * W + xpt
    w_ref[...] = wy * wx * valid * (1.0 / (SR * SR))


def _compute_idx_w(rois_pad, H, W):
    blk = 128
    grid = N_PAD // blk
    return pl.pallas_call(
        functools.partial(_idxw_body, H=H, W=W, HW=H * W),
        grid=(grid,),
        in_specs=[pl.BlockSpec((blk, 5), lambda i: (i, 0))],
        out_specs=[
            pl.BlockSpec((blk, J_PER_ROI), lambda i: (i, 0)),
            pl.BlockSpec((blk, J_PER_ROI), lambda i: (i, 0)),
        ],
        out_shape=[
            jax.ShapeDtypeStruct((N_PAD, J_PER_ROI), jnp.int32),
            jax.ShapeDtypeStruct((N_PAD, J_PER_ROI), jnp.float32),
        ],
    )(rois_pad)


def _pack_body(x_ref, o_ref):
    _, Cc, HB, Wd = x_ref.shape
    x = x_ref[0].reshape(Cc // 2, 2, HB * Wd)
    lo = x[:, 0, :]
    hi = x[:, 1, :]

    def rne16(v):
        u = lax.bitcast_convert_type(v, jnp.int32)
        return (u + 0x7FFF + ((u >> 16) & 1)) >> 16

    word = (rne16(lo) & 0xFFFF) | (rne16(hi) << 16)
    o_ref[0] = jnp.transpose(word, (1, 0)).reshape(HB, Wd, Cc // 2)


def _pack_table(feat):
    B, C, H, W = feat.shape
    HB = 8
    packed = pl.pallas_call(
        _pack_body,
        grid=(B, H // HB),
        in_specs=[pl.BlockSpec((1, C, HB, W), lambda b, h: (b, 0, h, 0))],
        out_specs=pl.BlockSpec((1, HB, W, C // 2), lambda b, h: (b, h, 0, 0)),
        out_shape=jax.ShapeDtypeStruct((B, H, W, C // 2), jnp.int32),
    )(feat)
    return packed.reshape(B * H * W, C // 2)


def _sc_body(idx_hbm, w_hbm, table_hbm, out_hbm,
             idx_v, w_v, rows0, rows1, out0, out1,
             sem_g0, sem_g1, sem_o0, sem_o1):
    nc = 2
    wid = lax.axis_index("s") * nc + lax.axis_index("c")
    base_bin = wid * BINS_PER_WORKER

    kpc = CHUNK_BINS * K_PER_BIN
    pltpu.sync_copy(idx_hbm.at[pl.ds(base_bin * K_PER_BIN, BINS_PER_WORKER * K_PER_BIN)], idx_v)
    pltpu.sync_copy(w_hbm.at[pl.ds(base_bin * K_PER_BIN, BINS_PER_WORKER * K_PER_BIN)], w_v)

    bufs = ((rows0, sem_g0, out0, sem_o0), (rows1, sem_g1, out1, sem_o1))

    pltpu.async_copy(table_hbm.at[idx_v.at[pl.ds(0, kpc)]], rows0, sem_g0)
    pltpu.async_copy(table_hbm.at[idx_v.at[pl.ds(kpc, kpc)]], rows1, sem_g1)

    def pair_body(g, _):
        for par in range(2):
            rows_v, sem_g, out_v, sem_o = bufs[par]
            ch = 2 * g + par
            bin0 = base_bin + ch * CHUNK_BINS
            pltpu.make_async_copy(table_hbm.at[idx_v.at[pl.ds(ch * kpc, kpc)]], rows_v, sem_g).wait()
            @pl.when(g > 0)
            def _():
                pltpu.make_async_copy(
                    out_v, out_hbm.at[pl.ds(bin0, CHUNK_BINS)], sem_o).wait()

            def bin_body(i, _):
                wv = w_v[pl.ds(ch * kpc + i * K_PER_BIN, K_PER_BIN)]
                acc = [jnp.zeros((16,), jnp.float32) for _ in range(16)]
                for k in range(K_PER_BIN):
                    wk = jnp.broadcast_to(wv[k], (16,))
                    r = i * K_PER_BIN + k
                    for g in range(8):
                        u = rows_v[r, pl.ds(g * 16, 16)]
                        a = lax.bitcast_convert_type(u << 16, jnp.float32)
                        b = lax.bitcast_convert_type(u, jnp.float32)
                        acc[2 * g] = acc[2 * g] + wk * a
                        acc[2 * g + 1] = acc[2 * g + 1] + wk * b
                for c in range(16):
                    out_v[i, pl.ds(c * 16, 16)] = acc[c]
                return 0

            lax.fori_loop(0, CHUNK_BINS, bin_body, 0)
            pltpu.async_copy(out_v, out_hbm.at[pl.ds(bin0, CHUNK_BINS)], sem_o)
            @pl.when(ch + 2 < CHUNKS_PER_WORKER)
            def _():
                pltpu.async_copy(table_hbm.at[idx_v.at[pl.ds((ch + 2) * kpc, kpc)]], rows_v, sem_g)
        return 0

    lax.fori_loop(0, CHUNKS_PER_WORKER // 2, pair_body, 0)
    last0 = base_bin + (CHUNKS_PER_WORKER - 2) * CHUNK_BINS
    last1 = base_bin + (CHUNKS_PER_WORKER - 1) * CHUNK_BINS
    pltpu.make_async_copy(out0, out_hbm.at[pl.ds(last0, CHUNK_BINS)], sem_o0).wait()
    pltpu.make_async_copy(out1, out_hbm.at[pl.ds(last1, CHUNK_BINS)], sem_o1).wait()


def _sc_gather(idx_flat, w_flat, table):
    C = table.shape[1] * 2
    mesh = plsc.VectorSubcoreMesh(core_axis_name="c", subcore_axis_name="s")
    run = pl.kernel(
        _sc_body,
        out_type=jax.ShapeDtypeStruct((TOTAL_BINS, C), jnp.float32),
        mesh=mesh,
        scratch_types=[
            pltpu.VMEM((BINS_PER_WORKER * K_PER_BIN,), jnp.int32),
            pltpu.VMEM((BINS_PER_WORKER * K_PER_BIN,), jnp.float32),
            pltpu.VMEM((CHUNK_BINS * K_PER_BIN, C // 2), jnp.int32),
            pltpu.VMEM((CHUNK_BINS * K_PER_BIN, C // 2), jnp.int32),
            pltpu.VMEM((CHUNK_BINS, C), jnp.float32),
            pltpu.VMEM((CHUNK_BINS, C), jnp.float32),
            pltpu.SemaphoreType.DMA,
            pltpu.SemaphoreType.DMA,
            pltpu.SemaphoreType.DMA,
            pltpu.SemaphoreType.DMA,
        ],
    )
    return run(idx_flat, w_flat, table)


def kernel(input, rois):
    B, C, H, W = input.shape
    N = rois.shape[0]
    nbins = OUT_H * OUT_W
    table = _pack_table(input)
    rois_pad = jnp.zeros((N_PAD, 5), jnp.float32).at[:N].set(rois.astype(jnp.float32))
    idx, w = _compute_idx_w(rois_pad, H, W)
    out_rows = _sc_gather(idx.reshape(-1), w.reshape(-1), table)
    out = out_rows.reshape(N_PAD, nbins, C // 32, 2, 16)
    out = jnp.transpose(out, (0, 2, 4, 3, 1)).reshape(N_PAD, C, nbins)
    return out[:N].reshape(N, C, OUT_H, OUT_W)


# speedup vs baseline: 1.4828x; 1.3162x over previous
import functools

import jax
import jax.numpy as jnp
from jax import lax
from jax.experimental import pallas as pl
from jax.experimental.pallas import tpu as pltpu
from jax.experimental.pallas import tpu_sc as plsc

OUT_H = 7
OUT_W = 7
SCALE = 0.25
SR = 2

K_PER_BIN = OUT_H * OUT_W * SR * SR * 4 // (OUT_H * OUT_W)
J_PER_ROI = OUT_H * OUT_W * K_PER_BIN

N_PAD = 1024
NUM_WORKERS = 32
TOTAL_BINS = N_PAD * OUT_H * OUT_W
BINS_PER_WORKER = TOTAL_BINS // NUM_WORKERS
CHUNK_BINS = 8
CHUNKS_PER_WORKER = BINS_PER_WORKER // CHUNK_BINS


def _idxw_body(rois_ref, idx_ref, w_ref, *, H, W, HW):
    blk = rois_ref.shape[0]
    j = lax.broadcasted_iota(jnp.int32, (blk, J_PER_ROI), 1)
    ax = j % 2
    ay = (j // 2) % 2
    sx = (j // 4) % 2
    sy = (j // 8) % 2
    q = (j // 16) % OUT_W
    p = j // (16 * OUT_W)

    bidx = rois_ref[:, 0:1].astype(jnp.int32)
    x1 = rois_ref[:, 1:2] * SCALE
    y1 = rois_ref[:, 2:3] * SCALE
    x2 = rois_ref[:, 3:4] * SCALE
    y2 = rois_ref[:, 4:5] * SCALE
    roi_w = jnp.maximum(x2 - x1, 1.0)
    roi_h = jnp.maximum(y2 - y1, 1.0)
    jy = (p * SR + sy).astype(jnp.float32)
    jx = (q * SR + sx).astype(jnp.float32)
    y = y1 + (jy + 0.5) * (roi_h / (OUT_H * SR))
    x = x1 + (jx + 0.5) * (roi_w / (OUT_W * SR))

    vy = (y >= -1.0) & (y <= float(H))
    vx = (x >= -1.0) & (x <= float(W))
    yc = jnp.clip(y, 0.0, float(H - 1))
    xc = jnp.clip(x, 0.0, float(W - 1))
    yl = jnp.floor(yc)
    xl = jnp.floor(xc)
    ly = yc - yl
    lx = xc - xl
    yl_i = yl.astype(jnp.int32)
    xl_i = xl.astype(jnp.int32)
    yh_i = jnp.minimum(yl_i + 1, H - 1)
    xh_i = jnp.minimum(xl_i + 1, W - 1)

    ypt = jnp.where(ay == 1, yh_i, yl_i)
    xpt = jnp.where(ax == 1, xh_i, xl_i)
    wy = jnp.where(ay == 1, ly, 1.0 - ly)
    wx = jnp.where(ax == 1, lx, 1.0 - lx)
    valid = (vy & vx).astype(jnp.float32)

    idx_ref[...] = bidx * HW + ypt * W + xpt
    w_ref[...] = wy * wx * valid * (1.0 / (SR * SR))


def _compute_idx_w(rois_pad, H, W):
    blk = 128
    grid = N_PAD // blk
    return pl.pallas_call(
        functools.partial(_idxw_body, H=H, W=W, HW=H * W),
        grid=(grid,),
        in_specs=[pl.BlockSpec((blk, 5), lambda i: (i, 0))],
        out_specs=[
            pl.BlockSpec((blk, J_PER_ROI), lambda i: (i, 0)),
            pl.BlockSpec((blk, J_PER_ROI), lambda i: (i, 0)),
        ],
        out_shape=[
            jax.ShapeDtypeStruct((N_PAD, J_PER_ROI), jnp.int32),
            jax.ShapeDtypeStruct((N_PAD, J_PER_ROI), jnp.float32),
        ],
    )(rois_pad)


def _pack_body(x_ref, o_ref):
    _, Cc, HB, Wd = x_ref.shape
    x = x_ref[0].reshape(Cc // 32, 2, 16, HB * Wd)
    lo = x[:, 0]
    hi = x[:, 1]

    def rne16(v):
        u = lax.bitcast_convert_type(v, jnp.int32)
        return (u + 0x7FFF + ((u >> 16) & 1)) >> 16

    word = ((rne16(lo) & 0xFFFF) | (rne16(hi) << 16)).reshape(Cc // 2, HB * Wd)
    o_ref[0] = jnp.transpose(word, (1, 0)).reshape(HB, Wd, Cc // 2)


def _pack_table(feat):
    B, C, H, W = feat.shape
    HB = 8
    packed = pl.pallas_call(
        _pack_body,
        grid=(B, H // HB),
        in_specs=[pl.BlockSpec((1, C, HB, W), lambda b, h: (b, 0, h, 0))],
        out_specs=pl.BlockSpec((1, HB, W, C // 2), lambda b, h: (b, h, 0, 0)),
        out_shape=jax.ShapeDtypeStruct((B, H, W, C // 2), jnp.int32),
    )(feat)
    return packed.reshape(B * H * W, C // 2)


def _sc_body(idx_hbm, w_hbm, table_hbm, out_hbm,
             idx_v, w_v, rows0, rows1, out0, out1,
             sem_g0, sem_g1, sem_o0, sem_o1):
    nc = 2
    wid = lax.axis_index("s") * nc + lax.axis_index("c")
    base_bin = wid * BINS_PER_WORKER

    kpc = CHUNK_BINS * K_PER_BIN
    pltpu.sync_copy(idx_hbm.at[pl.ds(base_bin * K_PER_BIN, BINS_PER_WORKER * K_PER_BIN)], idx_v)
    pltpu.sync_copy(w_hbm.at[pl.ds(base_bin * K_PER_BIN, BINS_PER_WORKER * K_PER_BIN)], w_v)

    bufs = ((rows0, sem_g0, out0, sem_o0), (rows1, sem_g1, out1, sem_o1))

    pltpu.async_copy(table_hbm.at[idx_v.at[pl.ds(0, kpc)]], rows0, sem_g0)
    pltpu.async_copy(table_hbm.at[idx_v.at[pl.ds(kpc, kpc)]], rows1, sem_g1)

    def pair_body(g, _):
        for par in range(2):
            rows_v, sem_g, out_v, sem_o = bufs[par]
            ch = 2 * g + par
            bin0 = base_bin + ch * CHUNK_BINS
            pltpu.make_async_copy(table_hbm.at[idx_v.at[pl.ds(ch * kpc, kpc)]], rows_v, sem_g).wait()
            @pl.when(g > 0)
            def _():
                pltpu.make_async_copy(
                    out_v, out_hbm.at[pl.ds(bin0, CHUNK_BINS)], sem_o).wait()

            def bin_body(i, _):
                wv = w_v[pl.ds(ch * kpc + i * K_PER_BIN, K_PER_BIN)]
                acc = [jnp.zeros((16,), jnp.float32) for _ in range(16)]
                for k in range(K_PER_BIN):
                    wk = jnp.broadcast_to(wv[k], (16,))
                    r = i * K_PER_BIN + k
                    for g in range(8):
                        u = rows_v[r, pl.ds(g * 16, 16)]
                        a = lax.bitcast_convert_type(u << 16, jnp.float32)
                        b = lax.bitcast_convert_type(u, jnp.float32)
                        acc[2 * g] = acc[2 * g] + wk * a
                        acc[2 * g + 1] = acc[2 * g + 1] + wk * b
                for c in range(16):
                    out_v[i, pl.ds(c * 16, 16)] = acc[c]
                return 0

            lax.fori_loop(0, CHUNK_BINS, bin_body, 0)
            pltpu.async_copy(out_v, out_hbm.at[pl.ds(bin0, CHUNK_BINS)], sem_o)
            @pl.when(ch + 2 < CHUNKS_PER_WORKER)
            def _():
                pltpu.async_copy(table_hbm.at[idx_v.at[pl.ds((ch + 2) * kpc, kpc)]], rows_v, sem_g)
        return 0

    lax.fori_loop(0, CHUNKS_PER_WORKER // 2, pair_body, 0)
    last0 = base_bin + (CHUNKS_PER_WORKER - 2) * CHUNK_BINS
    last1 = base_bin + (CHUNKS_PER_WORKER - 1) * CHUNK_BINS
    pltpu.make_async_copy(out0, out_hbm.at[pl.ds(last0, CHUNK_BINS)], sem_o0).wait()
    pltpu.make_async_copy(out1, out_hbm.at[pl.ds(last1, CHUNK_BINS)], sem_o1).wait()


def _sc_gather(idx_flat, w_flat, table):
    C = table.shape[1] * 2
    mesh = plsc.VectorSubcoreMesh(core_axis_name="c", subcore_axis_name="s")
    run = pl.kernel(
        _sc_body,
        out_type=jax.ShapeDtypeStruct((TOTAL_BINS, C), jnp.float32),
        mesh=mesh,
        scratch_types=[
            pltpu.VMEM((BINS_PER_WORKER * K_PER_BIN,), jnp.int32),
            pltpu.VMEM((BINS_PER_WORKER * K_PER_BIN,), jnp.float32),
            pltpu.VMEM((CHUNK_BINS * K_PER_BIN, C // 2), jnp.int32),
            pltpu.VMEM((CHUNK_BINS * K_PER_BIN, C // 2), jnp.int32),
            pltpu.VMEM((CHUNK_BINS, C), jnp.float32),
            pltpu.VMEM((CHUNK_BINS, C), jnp.float32),
            pltpu.SemaphoreType.DMA,
            pltpu.SemaphoreType.DMA,
            pltpu.SemaphoreType.DMA,
            pltpu.SemaphoreType.DMA,
        ],
    )
    return run(idx_flat, w_flat, table)


def kernel(input, rois):
    B, C, H, W = input.shape
    N = rois.shape[0]
    nbins = OUT_H * OUT_W
    table = _pack_table(input)
    rois_pad = jnp.zeros((N_PAD, 5), jnp.float32).at[:N].set(rois.astype(jnp.float32))
    idx, w = _compute_idx_w(rois_pad, H, W)
    out_rows = _sc_gather(idx.reshape(-1), w.reshape(-1), table)
    out = out_rows.reshape(N_PAD, nbins, C)
    out = jnp.transpose(out, (0, 2, 1))
    return out[:N].reshape(N, C, OUT_H, OUT_W)


# speedup vs baseline: 1.6574x; 1.1178x over previous
import functools

import jax
import jax.numpy as jnp
from jax import lax
from jax.experimental import pallas as pl
from jax.experimental.pallas import tpu as pltpu
from jax.experimental.pallas import tpu_sc as plsc

OUT_H = 7
OUT_W = 7
SCALE = 0.25
SR = 2

K_PER_BIN = OUT_H * OUT_W * SR * SR * 4 // (OUT_H * OUT_W)
J_PER_ROI = OUT_H * OUT_W * K_PER_BIN

N_PAD = 1024
NUM_WORKERS = 32
TOTAL_BINS = N_PAD * OUT_H * OUT_W
BINS_PER_WORKER = TOTAL_BINS // NUM_WORKERS
CHUNK_BINS = 8
CHUNKS_PER_WORKER = BINS_PER_WORKER // CHUNK_BINS


def _idxw_body(rois_ref, idx_ref, w_ref, *, H, W, HW):
    blk = rois_ref.shape[0]
    j = lax.broadcasted_iota(jnp.int32, (blk, J_PER_ROI), 1)
    ax = j % 2
    ay = (j // 2) % 2
    sx = (j // 4) % 2
    sy = (j // 8) % 2
    q = (j // 16) % OUT_W
    p = j // (16 * OUT_W)

    bidx = rois_ref[:, 0:1].astype(jnp.int32)
    x1 = rois_ref[:, 1:2] * SCALE
    y1 = rois_ref[:, 2:3] * SCALE
    x2 = rois_ref[:, 3:4] * SCALE
    y2 = rois_ref[:, 4:5] * SCALE
    roi_w = jnp.maximum(x2 - x1, 1.0)
    roi_h = jnp.maximum(y2 - y1, 1.0)
    jy = (p * SR + sy).astype(jnp.float32)
    jx = (q * SR + sx).astype(jnp.float32)
    y = y1 + (jy + 0.5) * (roi_h / (OUT_H * SR))
    x = x1 + (jx + 0.5) * (roi_w / (OUT_W * SR))

    vy = (y >= -1.0) & (y <= float(H))
    vx = (x >= -1.0) & (x <= float(W))
    yc = jnp.clip(y, 0.0, float(H - 1))
    xc = jnp.clip(x, 0.0, float(W - 1))
    yl = jnp.floor(yc)
    xl = jnp.floor(xc)
    ly = yc - yl
    lx = xc - xl
    yl_i = yl.astype(jnp.int32)
    xl_i = xl.astype(jnp.int32)
    yh_i = jnp.minimum(yl_i + 1, H - 1)
    xh_i = jnp.minimum(xl_i + 1, W - 1)

    ypt = jnp.where(ay == 1, yh_i, yl_i)
    xpt = jnp.where(ax == 1, xh_i, xl_i)
    wy = jnp.where(ay == 1, ly, 1.0 - ly)
    wx = jnp.where(ax == 1, lx, 1.0 - lx)
    valid = (vy & vx).astype(jnp.float32)

    idx_ref[...] = bidx * HW + ypt * W + xpt
    w_ref[...] = wy * wx * valid * (1.0 / (SR * SR))


def _compute_idx_w(rois_pad, H, W):
    blk = 128
    grid = N_PAD // blk
    return pl.pallas_call(
        functools.partial(_idxw_body, H=H, W=W, HW=H * W),
        grid=(grid,),
        in_specs=[pl.BlockSpec((blk, 5), lambda i: (i, 0))],
        out_specs=[
            pl.BlockSpec((blk, J_PER_ROI), lambda i: (i, 0)),
            pl.BlockSpec((blk, J_PER_ROI), lambda i: (i, 0)),
        ],
        out_shape=[
            jax.ShapeDtypeStruct((N_PAD, J_PER_ROI), jnp.int32),
            jax.ShapeDtypeStruct((N_PAD, J_PER_ROI), jnp.float32),
        ],
    )(rois_pad)


def _sc_body(idx_hbm, w_hbm, table_hbm, out_hbm,
             idx_v, w_v, rows0, rows1, out0, out1,
             sem_g0, sem_g1, sem_o0, sem_o1):
    nc = 2
    wid = lax.axis_index("s") * nc + lax.axis_index("c")
    base_bin = wid * BINS_PER_WORKER

    kpc = CHUNK_BINS * K_PER_BIN
    pltpu.sync_copy(idx_hbm.at[pl.ds(base_bin * K_PER_BIN, BINS_PER_WORKER * K_PER_BIN)], idx_v)
    pltpu.sync_copy(w_hbm.at[pl.ds(base_bin * K_PER_BIN, BINS_PER_WORKER * K_PER_BIN)], w_v)

    bufs = ((rows0, sem_g0, out0, sem_o0), (rows1, sem_g1, out1, sem_o1))

    pltpu.async_copy(table_hbm.at[idx_v.at[pl.ds(0, kpc)]], rows0, sem_g0)
    pltpu.async_copy(table_hbm.at[idx_v.at[pl.ds(kpc, kpc)]], rows1, sem_g1)

    def pair_body(g, _):
        for par in range(2):
            rows_v, sem_g, out_v, sem_o = bufs[par]
            ch = 2 * g + par
            bin0 = base_bin + ch * CHUNK_BINS
            pltpu.make_async_copy(table_hbm.at[idx_v.at[pl.ds(ch * kpc, kpc)]], rows_v, sem_g).wait()
            @pl.when(g > 0)
            def _():
                pltpu.make_async_copy(
                    out_v, out_hbm.at[pl.ds(bin0, CHUNK_BINS)], sem_o).wait()

            def bin_body(i, _):
                wv = w_v[pl.ds(ch * kpc + i * K_PER_BIN, K_PER_BIN)]
                acc = [jnp.zeros((16,), jnp.float32) for _ in range(16)]
                for k in range(K_PER_BIN):
                    wk = jnp.broadcast_to(wv[k], (16,))
                    r = i * K_PER_BIN + k
                    for c in range(16):
                        acc[c] = acc[c] + wk * rows_v[r, pl.ds(c * 16, 16)]
                for c in range(16):
                    out_v[i, pl.ds(c * 16, 16)] = acc[c]
                return 0

            lax.fori_loop(0, CHUNK_BINS, bin_body, 0)
            pltpu.async_copy(out_v, out_hbm.at[pl.ds(bin0, CHUNK_BINS)], sem_o)
            @pl.when(ch + 2 < CHUNKS_PER_WORKER)
            def _():
                pltpu.async_copy(table_hbm.at[idx_v.at[pl.ds((ch + 2) * kpc, kpc)]], rows_v, sem_g)
        return 0

    lax.fori_loop(0, CHUNKS_PER_WORKER // 2, pair_body, 0)
    last0 = base_bin + (CHUNKS_PER_WORKER - 2) * CHUNK_BINS
    last1 = base_bin + (CHUNKS_PER_WORKER - 1) * CHUNK_BINS
    pltpu.make_async_copy(out0, out_hbm.at[pl.ds(last0, CHUNK_BINS)], sem_o0).wait()
    pltpu.make_async_copy(out1, out_hbm.at[pl.ds(last1, CHUNK_BINS)], sem_o1).wait()


def _sc_gather(idx_flat, w_flat, table):
    C = table.shape[1]
    mesh = plsc.VectorSubcoreMesh(core_axis_name="c", subcore_axis_name="s")
    run = pl.kernel(
        _sc_body,
        out_type=jax.ShapeDtypeStruct((TOTAL_BINS, C), jnp.float32),
        mesh=mesh,
        scratch_types=[
            pltpu.VMEM((BINS_PER_WORKER * K_PER_BIN,), jnp.int32),
            pltpu.VMEM((BINS_PER_WORKER * K_PER_BIN,), jnp.float32),
            pltpu.VMEM((CHUNK_BINS * K_PER_BIN, C), jnp.float32),
            pltpu.VMEM((CHUNK_BINS * K_PER_BIN, C), jnp.float32),
            pltpu.VMEM((CHUNK_BINS, C), jnp.float32),
            pltpu.VMEM((CHUNK_BINS, C), jnp.float32),
            pltpu.SemaphoreType.DMA,
            pltpu.SemaphoreType.DMA,
            pltpu.SemaphoreType.DMA,
            pltpu.SemaphoreType.DMA,
        ],
    )
    return run(idx_flat, w_flat, table)


def kernel(input, rois):
    B, C, H, W = input.shape
    N = rois.shape[0]
    nbins = OUT_H * OUT_W
    table = jnp.transpose(input, (0, 2, 3, 1)).reshape(B * H * W, C)
    rois_pad = jnp.zeros((N_PAD, 5), jnp.float32).at[:N].set(rois.astype(jnp.float32))
    idx, w = _compute_idx_w(rois_pad, H, W)
    out_rows = _sc_gather(idx.reshape(-1), w.reshape(-1), table)
    out = out_rows.reshape(N_PAD, nbins, C)
    out = jnp.transpose(out, (0, 2, 1))
    return out[:N].reshape(N, C, OUT_H, OUT_W)
